# baseline (speedup 1.0000x reference)
import jax
import jax.numpy as jnp
from jax import lax
from jax.experimental import pallas as pl
from jax.experimental.pallas import tpu as pltpu
from jax.experimental.pallas import tpu_sc as plsc

NC = 2
NS = 16
NW = NC * NS
L = 16

D = 128
S = 4096
T = 50
TP = 56
S_PER_W = S // NW
R_PER_W = S_PER_W * TP
GL = 112
G_PER_CH = 4
CH_ROWS = GL * G_PER_CH
NCHUNK = R_PER_W // CH_ROWS
IDX_ROWS = R_PER_W // GL


def _worker_body(table, xp, zrow, out, idx_v, buf0, buf1, g0, g1, s0, s1):
    wid = lax.axis_index("s") * NC + lax.axis_index("c")
    row0 = wid * R_PER_W
    bufs = (buf0, buf1)
    gsems = (g0, g1)
    ssems = (s0, s1)

    pltpu.sync_copy(xp.at[wid], idx_v)

    def gather_parts(c, k):
        for j in range(G_PER_CH):
            yield (table.at[idx_v.at[c * G_PER_CH + j]],
                   bufs[k].at[pl.ds(j * GL, GL)], gsems[k])

    def start_gather(c, k):
        for src, dst, sem in gather_parts(c, k):
            pltpu.async_copy(src, dst, sem)

    def wait_gather(c, k):
        for src, dst, sem in gather_parts(c, k):
            pltpu.make_async_copy(src, dst, sem).wait()

    def out_slice(c):
        return out.at[pl.ds(row0 + c * CH_ROWS, CH_ROWS)]

    def start_scatter(c, k):
        pltpu.async_copy(bufs[k], out_slice(c), ssems[k])

    def wait_scatter(c, k):
        pltpu.make_async_copy(bufs[k], out_slice(c), ssems[k]).wait()

    def process(c, k):
        wait_gather(c, k)
        start_scatter(c, k)

    start_gather(0, 0)
    process(0, 0)
    start_gather(1, 1)

    @pl.loop(0, (NCHUNK - 2) // 2)
    def _steady(i):
        c1 = 2 * i + 1
        process(c1, 1)
        wait_scatter(c1 - 1, 0)
        start_gather(c1 + 1, 0)
        c2 = 2 * i + 2
        process(c2, 0)
        wait_scatter(c2 - 1, 1)
        start_gather(c2 + 1, 1)

    process(NCHUNK - 1, 1)
    wait_scatter(NCHUNK - 2, 0)
    wait_scatter(NCHUNK - 1, 1)

    @pl.loop(0, IDX_ROWS)
    def _fix_block(bl):
        vs = [idx_v[bl, pl.ds(g * L, L)] for g in range(GL // L)]
        bmn = vs[0]
        for v in vs[1:]:
            bmn = jnp.minimum(bmn, v)
        blk_pad = plsc.all_reduce_population_count(bmn == 0)[0] > 0

        @pl.when(blk_pad)
        def _patch():
            for g in range(GL // L):
                for lane in range(L):
                    off = g * L + lane

                    @pl.when(vs[g][lane] == 0)
                    def _zero_row():
                        pltpu.sync_copy(zrow.at[0],
                                        out.at[row0 + bl * GL + off])


@jax.jit
def kernel(embedding, x):
    xp = jnp.full((S, TP), 1, jnp.int32)
    xp = xp.at[:, :T].set(x.astype(jnp.int32))
    xp = xp.reshape(NW, IDX_ROWS, GL)
    zrow = jnp.zeros((8, D), jnp.float32)
    mesh = plsc.VectorSubcoreMesh(
        core_axis_name="c", subcore_axis_name="s",
        num_cores=NC, num_subcores=NS,
    )
    out = pl.kernel(
        _worker_body,
        out_type=jax.ShapeDtypeStruct((S * TP, D), jnp.float32),
        mesh=mesh,
        compiler_params=pltpu.CompilerParams(needs_layout_passes=False),
        scratch_types=[
            pltpu.VMEM((IDX_ROWS, GL), jnp.int32),
            pltpu.VMEM((CH_ROWS, D), jnp.float32),
            pltpu.VMEM((CH_ROWS, D), jnp.float32),
            pltpu.SemaphoreType.DMA,
            pltpu.SemaphoreType.DMA,
            pltpu.SemaphoreType.DMA,
            pltpu.SemaphoreType.DMA,
        ],
    )(embedding, xp, zrow)
    return out.reshape(S, TP, D)[:, :T, :]


# speedup vs baseline: 7.5997x; 7.5997x over previous
import jax
import jax.numpy as jnp
from jax import lax
from jax.experimental import pallas as pl
from jax.experimental.pallas import tpu as pltpu
from jax.experimental.pallas import tpu_sc as plsc

NC = 2
NS = 16
NW = NC * NS
L = 16

D = 128
S = 4096
T = 50
S_PER_W = S // NW
CH_S = 4
NBUF = 4
PREF = 2
NCHUNK = S_PER_W // CH_S

_GROUP_OFF = (0, 16, 32, T - L)


def _worker_body(table, x, zrow, out, idx_v, bufs, gsems, ssems):
    wid = lax.axis_index("s") * NC + lax.axis_index("c")
    sent0 = wid * S_PER_W

    pltpu.sync_copy(x.at[pl.ds(sent0, S_PER_W)], idx_v)

    def gather_parts(c, k):
        for j in range(CH_S):
            yield (table.at[idx_v.at[c * CH_S + j]], bufs[k].at[j], gsems[k])

    def start_gather(c, k):
        for src, dst, sem in gather_parts(c, k):
            pltpu.async_copy(src, dst, sem)

    def wait_gather(c, k):
        for src, dst, sem in gather_parts(c, k):
            pltpu.make_async_copy(src, dst, sem).wait()

    def out_slice(c):
        return out.at[pl.ds(sent0 + c * CH_S, CH_S)]

    def start_scatter(c, k):
        pltpu.async_copy(bufs[k], out_slice(c), ssems[k])

    def wait_scatter(c, k):
        pltpu.make_async_copy(bufs[k], out_slice(c), ssems[k]).wait()

    for p in range(PREF):
        start_gather(p, p % NBUF)

    @pl.loop(0, NCHUNK // NBUF)
    def _steady(i):
        for t in range(NBUF):
            c = i * NBUF + t
            wait_gather(c, t)
            start_scatter(c, t)
            p = c + PREF
            kp = (t + PREF) % NBUF

            @pl.when(c >= NBUF - PREF)
            def _drain():
                wait_scatter(c - (NBUF - PREF), kp)

            @pl.when(p < NCHUNK)
            def _prefetch():
                start_gather(p, kp)

    for c in range(NCHUNK - (NBUF - PREF), NCHUNK):
        wait_scatter(c, c % NBUF)

    @pl.loop(0, S_PER_W)
    def _fix_sent(sl):
        vs = [idx_v[sl, pl.ds(off, L)] for off in _GROUP_OFF]
        smn = vs[0]
        for v in vs[1:]:
            smn = jnp.minimum(smn, v)
        sent_pad = plsc.all_reduce_population_count(smn == 0)[0] > 0

        @pl.when(sent_pad)
        def _patch():
            for g, off in enumerate(_GROUP_OFF):
                for lane in range(L):
                    row = off + lane

                    @pl.when(vs[g][lane] == 0)
                    def _zero_row():
                        pltpu.sync_copy(zrow.at[0], out.at[sent0 + sl, row])


def _body(table, x, zrow, out, idx_v, b0, b1, b2, b3, g0, g1, g2, g3,
          s0, s1, s2, s3):
    _worker_body(table, x, zrow, out, idx_v, (b0, b1, b2, b3),
                 (g0, g1, g2, g3), (s0, s1, s2, s3))


@jax.jit
def kernel(embedding, x):
    xi = x.astype(jnp.int32)
    zrow = jnp.zeros((8, D), jnp.float32)
    mesh = plsc.VectorSubcoreMesh(
        core_axis_name="c", subcore_axis_name="s",
        num_cores=NC, num_subcores=NS,
    )
    return pl.kernel(
        _body,
        out_type=jax.ShapeDtypeStruct((S, T, D), jnp.float32),
        mesh=mesh,
        compiler_params=pltpu.CompilerParams(needs_layout_passes=False),
        scratch_types=(
            [pltpu.VMEM((S_PER_W, T), jnp.int32)]
            + [pltpu.VMEM((CH_S, T, D), jnp.float32) for _ in range(NBUF)]
            + [pltpu.SemaphoreType.DMA for _ in range(2 * NBUF)]
        ),
    )(embedding, xi, zrow)


# speedup vs baseline: 7.6036x; 1.0005x over previous
import jax
import jax.numpy as jnp
from jax import lax
from jax.experimental import pallas as pl
from jax.experimental.pallas import tpu as pltpu
from jax.experimental.pallas import tpu_sc as plsc

NC = 2
NS = 16
NW = NC * NS
L = 16

D = 128
S = 4096
T = 50
S_PER_W = S // NW
CH_S = 2
NBUF = 8
PREF = 4
NCHUNK = S_PER_W // CH_S

_GROUP_OFF = (0, 16, 32, T - L)


def _worker_body(table, x, zrow, out, idx_v, bufs, gsems, ssems):
    wid = lax.axis_index("s") * NC + lax.axis_index("c")
    sent0 = wid * S_PER_W

    pltpu.sync_copy(x.at[pl.ds(sent0, S_PER_W)], idx_v)

    def gather_parts(c, k):
        for j in range(CH_S):
            yield (table.at[idx_v.at[c * CH_S + j]], bufs[k].at[j], gsems[k])

    def start_gather(c, k):
        for src, dst, sem in gather_parts(c, k):
            pltpu.async_copy(src, dst, sem)

    def wait_gather(c, k):
        for src, dst, sem in gather_parts(c, k):
            pltpu.make_async_copy(src, dst, sem).wait()

    def out_slice(c):
        return out.at[pl.ds(sent0 + c * CH_S, CH_S)]

    def start_scatter(c, k):
        pltpu.async_copy(bufs[k], out_slice(c), ssems[k])

    def wait_scatter(c, k):
        pltpu.make_async_copy(bufs[k], out_slice(c), ssems[k]).wait()

    for p in range(PREF):
        start_gather(p, p % NBUF)

    @pl.loop(0, NCHUNK // NBUF)
    def _steady(i):
        for t in range(NBUF):
            c = i * NBUF + t
            wait_gather(c, t)
            start_scatter(c, t)
            p = c + PREF
            kp = (t + PREF) % NBUF

            @pl.when(c >= NBUF - PREF)
            def _drain():
                wait_scatter(c - (NBUF - PREF), kp)

            @pl.when(p < NCHUNK)
            def _prefetch():
                start_gather(p, kp)

    for c in range(NCHUNK - (NBUF - PREF), NCHUNK):
        wait_scatter(c, c % NBUF)

    @pl.loop(0, S_PER_W)
    def _fix_sent(sl):
        vs = [idx_v[sl, pl.ds(off, L)] for off in _GROUP_OFF]
        smn = vs[0]
        for v in vs[1:]:
            smn = jnp.minimum(smn, v)
        sent_pad = plsc.all_reduce_population_count(smn == 0)[0] > 0

        @pl.when(sent_pad)
        def _patch():
            for g, off in enumerate(_GROUP_OFF):
                for lane in range(L):
                    row = off + lane

                    @pl.when(vs[g][lane] == 0)
                    def _zero_row():
                        pltpu.sync_copy(zrow.at[0], out.at[sent0 + sl, row])


def _body(table, x, zrow, out, idx_v, *rest):
    _worker_body(table, x, zrow, out, idx_v, rest[:NBUF],
                 rest[NBUF:2 * NBUF], rest[2 * NBUF:])


@jax.jit
def kernel(embedding, x):
    xi = x.astype(jnp.int32)
    zrow = jnp.zeros((8, D), jnp.float32)
    mesh = plsc.VectorSubcoreMesh(
        core_axis_name="c", subcore_axis_name="s",
        num_cores=NC, num_subcores=NS,
    )
    return pl.kernel(
        _body,
        out_type=jax.ShapeDtypeStruct((S, T, D), jnp.float32),
        mesh=mesh,
        compiler_params=pltpu.CompilerParams(needs_layout_passes=False),
        scratch_types=(
            [pltpu.VMEM((S_PER_W, T), jnp.int32)]
            + [pltpu.VMEM((CH_S, T, D), jnp.float32) for _ in range(NBUF)]
            + [pltpu.SemaphoreType.DMA for _ in range(2 * NBUF)]
        ),
    )(embedding, xi, zrow)
